# baseline (speedup 1.0000x reference)
import functools

import jax
import jax.numpy as jnp
from jax.experimental import pallas as pl

B = 512
L = 128
F = 50
D = 64
H = 32
E = 8
VOCAB = 901
HEAD = 32
BT = 128
TC = 16
NEG = -3.0e38


def _stage1_kernel(x_ref, hor_ref, W_in_ref, b_in_ref, emb_ref, W_gate_ref,
                   b_gate_ref, xp_ref, wf_ref):
    hor = hor_ref[...].astype(jnp.int32)
    iota_v = jax.lax.broadcasted_iota(jnp.int32, (BT, VOCAB), 1)
    oh = (hor[:, None] == iota_v).astype(jnp.float32)
    he = jnp.dot(oh, emb_ref[...],
                 preferred_element_type=jnp.float32)

    xt = x_ref[...].reshape(BT * L, F)
    xp = jax.lax.dot_general(xt, W_in_ref[...], (((1,), (1,)), ((), ())),
                             preferred_element_type=jnp.float32)
    xp = xp + b_in_ref[...][None, :]
    xp = xp.reshape(BT, L, D) + he[:, None, :]
    xp_ref[...] = xp

    logits = jax.lax.dot_general(he, W_gate_ref[...], (((1,), (1,)), ((), ())),
                                 preferred_element_type=jnp.float32)
    logits = logits + b_gate_ref[...][None, :]
    iota_e = jax.lax.broadcasted_iota(jnp.int32, (BT, E), 1)
    m1 = jnp.max(logits, axis=1, keepdims=True)
    is1 = (logits == m1)
    idx1 = jnp.min(jnp.where(is1, iota_e, E), axis=1, keepdims=True)
    masked = jnp.where(iota_e == idx1, NEG, logits)
    m2 = jnp.max(masked, axis=1, keepdims=True)
    is2 = (masked == m2)
    idx2 = jnp.min(jnp.where(is2, iota_e, E), axis=1, keepdims=True)
    w1 = 1.0 / (1.0 + jnp.exp(m2 - m1))
    w2 = 1.0 - w1
    wf = jnp.where(iota_e == idx1, w1,
                   jnp.where(iota_e == idx2, w2, 0.0))
    wf_ref[...] = wf.T


def _gru_gates(gi, gh, h):
    r = jax.nn.sigmoid(gi[:, :H] + gh[:, :H])
    z = jax.nn.sigmoid(gi[:, H:2 * H] + gh[:, H:2 * H])
    n = jnp.tanh(gi[:, 2 * H:] + r * gh[:, 2 * H:])
    return (1.0 - z) * n + z * h


def _stage2_kernel(xp_ref, wf_ref, Wih0_ref, Whh0_ref, bih0_ref, bhh0_ref,
                   Wih1_ref, Whh1_ref, bih1_ref, bhh1_ref, Wh1_ref, bh1_ref,
                   Wh2_ref, bh2_ref, out_ref):
    e = pl.program_id(0)
    Wih0 = Wih0_ref[0]
    Whh0 = Whh0_ref[0]
    bih0 = bih0_ref[0]
    bhh0 = bhh0_ref[0]
    Wih1 = Wih1_ref[0]
    Whh1 = Whh1_ref[0]
    bih1 = bih1_ref[0]
    bhh1 = bhh1_ref[0]

    def chunk_body(c, carry):
        h0, h1 = carry
        xc = xp_ref[:, pl.ds(c * TC, TC), :]
        gi0c = jax.lax.dot_general(
            xc.reshape(B * TC, D), Wih0, (((1,), (1,)), ((), ())),
            preferred_element_type=jnp.float32)
        gi0c = gi0c.reshape(B, TC, 3 * H)

        for i in range(TC):
            gi0 = gi0c[:, i, :] + bih0
            gh0 = jax.lax.dot_general(h0, Whh0, (((1,), (1,)), ((), ())),
                                      preferred_element_type=jnp.float32)
            h0 = _gru_gates(gi0, gh0 + bhh0, h0)
            gi1 = jax.lax.dot_general(h0, Wih1, (((1,), (1,)), ((), ())),
                                      preferred_element_type=jnp.float32)
            gh1 = jax.lax.dot_general(h1, Whh1, (((1,), (1,)), ((), ())),
                                      preferred_element_type=jnp.float32)
            h1 = _gru_gates(gi1 + bih1, gh1 + bhh1, h1)
        return (h0, h1)

    h0 = jnp.zeros((B, H), jnp.float32)
    h1 = jnp.zeros((B, H), jnp.float32)
    h0, h1 = jax.lax.fori_loop(0, L // TC, chunk_body, (h0, h1))

    zh = jax.lax.dot_general(h1, Wh1_ref[0], (((1,), (1,)), ((), ())),
                             preferred_element_type=jnp.float32)
    zh = jnp.maximum(zh + bh1_ref[0], 0.0)
    pred = jax.lax.dot_general(zh, Wh2_ref[0], (((1,), (1,)), ((), ())),
                               preferred_element_type=jnp.float32)
    pred = pred.reshape(1, B) + bh2_ref[0, 0, 0]

    w_e = wf_ref[pl.ds(e, 1), :]
    contrib = w_e * pred

    @pl.when(e == 0)
    def _():
        out_ref[...] = contrib

    @pl.when(e != 0)
    def _():
        out_ref[...] += contrib


@jax.jit
def kernel(x, horizon, W_in, b_in, emb, W_gate, b_gate, W_ih0, W_hh0, b_ih0,
           b_hh0, W_ih1, W_hh1, b_ih1, b_hh1, W_h1, b_h1, W_h2, b_h2):
    x = x.astype(jnp.float32)
    horizon = horizon.astype(jnp.int32)

    xp, wf = pl.pallas_call(
        _stage1_kernel,
        grid=(B // BT,),
        in_specs=[
            pl.BlockSpec((BT, L, F), lambda i: (i, 0, 0)),
            pl.BlockSpec((BT,), lambda i: (i,)),
            pl.BlockSpec((D, F), lambda i: (0, 0)),
            pl.BlockSpec((D,), lambda i: (0,)),
            pl.BlockSpec((VOCAB, D), lambda i: (0, 0)),
            pl.BlockSpec((E, D), lambda i: (0, 0)),
            pl.BlockSpec((E,), lambda i: (0,)),
        ],
        out_specs=[
            pl.BlockSpec((BT, L, D), lambda i: (i, 0, 0)),
            pl.BlockSpec((E, BT), lambda i: (0, i)),
        ],
        out_shape=[
            jax.ShapeDtypeStruct((B, L, D), jnp.float32),
            jax.ShapeDtypeStruct((E, B), jnp.float32),
        ],
    )(x, horizon, W_in, b_in, emb, W_gate, b_gate)

    out = pl.pallas_call(
        _stage2_kernel,
        grid=(E,),
        in_specs=[
            pl.BlockSpec((B, L, D), lambda e: (0, 0, 0)),
            pl.BlockSpec((E, B), lambda e: (0, 0)),
            pl.BlockSpec((1, 3 * H, D), lambda e: (e, 0, 0)),
            pl.BlockSpec((1, 3 * H, H), lambda e: (e, 0, 0)),
            pl.BlockSpec((1, 1, 3 * H), lambda e: (e, 0, 0)),
            pl.BlockSpec((1, 1, 3 * H), lambda e: (e, 0, 0)),
            pl.BlockSpec((1, 3 * H, H), lambda e: (e, 0, 0)),
            pl.BlockSpec((1, 3 * H, H), lambda e: (e, 0, 0)),
            pl.BlockSpec((1, 1, 3 * H), lambda e: (e, 0, 0)),
            pl.BlockSpec((1, 1, 3 * H), lambda e: (e, 0, 0)),
            pl.BlockSpec((1, HEAD, H), lambda e: (e, 0, 0)),
            pl.BlockSpec((1, 1, HEAD), lambda e: (e, 0, 0)),
            pl.BlockSpec((1, 1, HEAD), lambda e: (e, 0, 0)),
            pl.BlockSpec((1, 1, 1), lambda e: (e, 0, 0)),
        ],
        out_specs=pl.BlockSpec((1, B), lambda e: (0, 0)),
        out_shape=jax.ShapeDtypeStruct((1, B), jnp.float32),
    )(xp, wf, W_ih0, W_hh0, b_ih0.reshape(E, 1, 3 * H),
      b_hh0.reshape(E, 1, 3 * H), W_ih1, W_hh1, b_ih1.reshape(E, 1, 3 * H),
      b_hh1.reshape(E, 1, 3 * H), W_h1, b_h1.reshape(E, 1, HEAD),
      W_h2, b_h2.reshape(E, 1, 1))

    return out[0]


# speedup vs baseline: 4.8455x; 4.8455x over previous
import jax
import jax.numpy as jnp
from jax.experimental import pallas as pl
from jax.experimental.pallas import tpu as pltpu

B = 512
L = 128
F = 50
D = 64
H = 32
E = 8
R = 2 * B
VOCAB = 901
HEAD = 32
BT = 128
TC = 16
NCH = L // TC
NEG = -3.0e38


def _stage1_kernel(x_ref, hor_ref, W_in_ref, b_in_ref, emb_ref, W_gate_ref,
                   b_gate_ref, xp_ref, eid_ref, ws_ref):
    hor = hor_ref[...].astype(jnp.int32)
    iota_v = jax.lax.broadcasted_iota(jnp.int32, (BT, VOCAB), 1)
    oh = (hor[:, None] == iota_v).astype(jnp.float32)
    he = jnp.dot(oh, emb_ref[...],
                 preferred_element_type=jnp.float32)

    xt = x_ref[...].reshape(BT * L, F)
    xp = jax.lax.dot_general(xt, W_in_ref[...], (((1,), (1,)), ((), ())),
                             preferred_element_type=jnp.float32)
    xp = xp + b_in_ref[...][None, :]
    xp = xp.reshape(BT, L, D) + he[:, None, :]
    xp_ref[...] = xp

    logits = jax.lax.dot_general(he, W_gate_ref[...], (((1,), (1,)), ((), ())),
                                 preferred_element_type=jnp.float32)
    logits = logits + b_gate_ref[...][None, :]
    iota_e = jax.lax.broadcasted_iota(jnp.int32, (BT, E), 1)
    m1 = jnp.max(logits, axis=1, keepdims=True)
    is1 = (logits == m1)
    idx1 = jnp.min(jnp.where(is1, iota_e, E), axis=1, keepdims=True)
    masked = jnp.where(iota_e == idx1, NEG, logits)
    m2 = jnp.max(masked, axis=1, keepdims=True)
    is2 = (masked == m2)
    idx2 = jnp.min(jnp.where(is2, iota_e, E), axis=1, keepdims=True)
    w1 = 1.0 / (1.0 + jnp.exp(m2 - m1))
    w2 = 1.0 - w1
    eid_ref[...] = jnp.concatenate([idx1[None], idx2[None]], axis=0)
    ws_ref[...] = jnp.concatenate([w1[None], w2[None]], axis=0)


def _tile8(v):
    return jnp.concatenate([v] * 8, axis=1)


def _stage2_kernel(xp_ref, eid_ref, ws_ref, Wih0T_ref, Whh0s_ref, bih0_ref,
                   bhh0_ref, W1s_ref, b1_ref, Wh1s_ref, bh1_ref, Wh2_ref,
                   bh2_ref, out_ref, h0_ref, h1_ref):
    c = pl.program_id(0)
    eidc = eid_ref[...]
    oh_e = (jax.lax.broadcasted_iota(jnp.int32, (R, E), 1)
            == eidc).astype(jnp.float32)
    m0 = (jax.lax.broadcasted_iota(jnp.int32, (R, E * H), 1) // H
          == eidc).astype(jnp.float32)
    m1 = (jax.lax.broadcasted_iota(jnp.int32, (R, E * 2 * H), 1) // (2 * H)
          == eidc).astype(jnp.float32)

    b0i = jnp.dot(oh_e, bih0_ref[...], preferred_element_type=jnp.float32)
    b0h = jnp.dot(oh_e, bhh0_ref[...], preferred_element_type=jnp.float32)
    b1r = jnp.dot(oh_e, b1_ref[...], preferred_element_type=jnp.float32)
    bh1r = jnp.dot(oh_e, bh1_ref[...], preferred_element_type=jnp.float32)
    wh2r = jnp.dot(oh_e, Wh2_ref[...], preferred_element_type=jnp.float32)
    bh2r = jnp.dot(oh_e, bh2_ref[...], preferred_element_type=jnp.float32)

    sel0 = [oh_e[:B, e].reshape(B, 1, 1) for e in range(E)]
    sel1 = [oh_e[B:, e].reshape(B, 1, 1) for e in range(E)]

    Whh0s = Whh0s_ref[...]
    W1s = W1s_ref[...]

    @pl.when(c == 0)
    def _():
        h0_ref[...] = jnp.zeros((R, H), jnp.float32)
        h1_ref[...] = jnp.zeros((R, H), jnp.float32)

    h0 = h0_ref[...]
    h1 = h1_ref[...]

    xf = xp_ref[...].reshape(B * TC, D)
    gs0 = jnp.zeros((B, TC, 3 * H), jnp.float32)
    gs1 = jnp.zeros((B, TC, 3 * H), jnp.float32)
    for e in range(E):
        ge = jnp.dot(xf, Wih0T_ref[e],
                     preferred_element_type=jnp.float32)
        ge = ge.reshape(B, TC, 3 * H)
        gs0 = gs0 + ge * sel0[e]
        gs1 = gs1 + ge * sel1[e]

    for i in range(TC):
        gi0 = jnp.concatenate([gs0[:, i, :], gs1[:, i, :]], axis=0)
        gi0 = gi0 + b0i
        gh0 = jnp.dot(_tile8(h0) * m0, Whh0s,
                      preferred_element_type=jnp.float32) + b0h
        r = jax.nn.sigmoid(gi0[:, :H] + gh0[:, :H])
        z = jax.nn.sigmoid(gi0[:, H:2 * H] + gh0[:, H:2 * H])
        n = jnp.tanh(gi0[:, 2 * H:] + r * gh0[:, 2 * H:])
        h0 = (1.0 - z) * n + z * h0

        cat1 = jnp.concatenate([h0, h1], axis=1)
        g1 = jnp.dot(_tile8(cat1) * m1, W1s,
                     preferred_element_type=jnp.float32) + b1r
        r1 = jax.nn.sigmoid(g1[:, :H])
        z1 = jax.nn.sigmoid(g1[:, H:2 * H])
        n1 = jnp.tanh(g1[:, 2 * H:3 * H] + r1 * g1[:, 3 * H:])
        h1 = (1.0 - z1) * n1 + z1 * h1

    h0_ref[...] = h0
    h1_ref[...] = h1

    @pl.when(c == NCH - 1)
    def _():
        zh = jnp.dot(_tile8(h1) * m0, Wh1s_ref[...],
                     preferred_element_type=jnp.float32) + bh1r
        zh_r = jnp.maximum(zh, 0.0)
        pred = jnp.sum(zh_r * wh2r, axis=1, keepdims=True) + bh2r
        ws = ws_ref[...]
        out_ref[...] = (ws[:B] * pred[:B]) + (ws[B:] * pred[B:])


@jax.jit
def kernel(x, horizon, W_in, b_in, emb, W_gate, b_gate, W_ih0, W_hh0, b_ih0,
           b_hh0, W_ih1, W_hh1, b_ih1, b_hh1, W_h1, b_h1, W_h2, b_h2):
    x = x.astype(jnp.float32)
    horizon = horizon.astype(jnp.int32)

    xp, eid, ws = pl.pallas_call(
        _stage1_kernel,
        grid=(B // BT,),
        in_specs=[
            pl.BlockSpec((BT, L, F), lambda i: (i, 0, 0)),
            pl.BlockSpec((BT,), lambda i: (i,)),
            pl.BlockSpec((D, F), lambda i: (0, 0)),
            pl.BlockSpec((D,), lambda i: (0,)),
            pl.BlockSpec((VOCAB, D), lambda i: (0, 0)),
            pl.BlockSpec((E, D), lambda i: (0, 0)),
            pl.BlockSpec((E,), lambda i: (0,)),
        ],
        out_specs=[
            pl.BlockSpec((BT, L, D), lambda i: (i, 0, 0)),
            pl.BlockSpec((2, BT, 1), lambda i: (0, i, 0)),
            pl.BlockSpec((2, BT, 1), lambda i: (0, i, 0)),
        ],
        out_shape=[
            jax.ShapeDtypeStruct((B, L, D), jnp.float32),
            jax.ShapeDtypeStruct((2, B, 1), jnp.int32),
            jax.ShapeDtypeStruct((2, B, 1), jnp.float32),
        ],
    )(x, horizon, W_in, b_in, emb, W_gate, b_gate)

    Wih0T = W_ih0.transpose(0, 2, 1)
    Whh0s = W_hh0.transpose(0, 2, 1).reshape(E * H, 3 * H)
    Wih1T = W_ih1.transpose(0, 2, 1)
    Whh1T = W_hh1.transpose(0, 2, 1)
    zH = jnp.zeros((E, H, H), jnp.float32)
    top = jnp.concatenate([Wih1T[:, :, :2 * H], Wih1T[:, :, 2 * H:], zH], 2)
    bot = jnp.concatenate([Whh1T[:, :, :2 * H], zH, Whh1T[:, :, 2 * H:]], 2)
    W1s = jnp.concatenate([top, bot], axis=1).reshape(E * 2 * H, 4 * H)
    b1 = jnp.concatenate([b_ih1[:, :2 * H] + b_hh1[:, :2 * H],
                          b_ih1[:, 2 * H:], b_hh1[:, 2 * H:]], axis=1)
    Wh1s = W_h1.transpose(0, 2, 1).reshape(E * H, HEAD)

    out = pl.pallas_call(
        _stage2_kernel,
        grid=(NCH,),
        in_specs=[
            pl.BlockSpec((B, TC, D), lambda c: (0, c, 0)),
            pl.BlockSpec((R, 1), lambda c: (0, 0)),
            pl.BlockSpec((R, 1), lambda c: (0, 0)),
            pl.BlockSpec((E, D, 3 * H), lambda c: (0, 0, 0)),
            pl.BlockSpec((E * H, 3 * H), lambda c: (0, 0)),
            pl.BlockSpec((E, 3 * H), lambda c: (0, 0)),
            pl.BlockSpec((E, 3 * H), lambda c: (0, 0)),
            pl.BlockSpec((E * 2 * H, 4 * H), lambda c: (0, 0)),
            pl.BlockSpec((E, 4 * H), lambda c: (0, 0)),
            pl.BlockSpec((E * H, HEAD), lambda c: (0, 0)),
            pl.BlockSpec((E, HEAD), lambda c: (0, 0)),
            pl.BlockSpec((E, HEAD), lambda c: (0, 0)),
            pl.BlockSpec((E, 1), lambda c: (0, 0)),
        ],
        out_specs=pl.BlockSpec((B, 1), lambda c: (0, 0)),
        out_shape=jax.ShapeDtypeStruct((B, 1), jnp.float32),
        scratch_shapes=[pltpu.VMEM((R, H), jnp.float32),
                        pltpu.VMEM((R, H), jnp.float32)],
    )(xp, eid.reshape(R, 1), ws.reshape(R, 1), Wih0T, Whh0s, b_ih0, b_hh0,
      W1s, b1, Wh1s, b_h1, W_h2.reshape(E, HEAD), b_h2)

    return out[:, 0]


# speedup vs baseline: 5.3220x; 1.0983x over previous
import jax
import jax.numpy as jnp
from jax.experimental import pallas as pl
from jax.experimental.pallas import tpu as pltpu

B = 512
L = 128
F = 50
D = 64
H = 32
E = 8
R = 2 * B
VOCAB = 901
HEAD = 32
BT = 128
TC = 16
NCH = L // TC
NEG = -3.0e38


def _stage1_kernel(x_ref, hor_ref, W_in_ref, b_in_ref, emb_ref, W_gate_ref,
                   b_gate_ref, xp_ref, eid_ref, ws_ref):
    hor = hor_ref[...].astype(jnp.int32)
    iota_v = jax.lax.broadcasted_iota(jnp.int32, (BT, VOCAB), 1)
    oh = (hor[:, None] == iota_v).astype(jnp.float32)
    he = jnp.dot(oh, emb_ref[...],
                 preferred_element_type=jnp.float32)

    xt = x_ref[...].reshape(BT * L, F)
    xp = jax.lax.dot_general(xt, W_in_ref[...], (((1,), (1,)), ((), ())),
                             preferred_element_type=jnp.float32)
    xp = xp + b_in_ref[...][None, :]
    xp = xp.reshape(BT, L, D) + he[:, None, :]
    xp_ref[...] = xp

    logits = jax.lax.dot_general(he, W_gate_ref[...], (((1,), (1,)), ((), ())),
                                 preferred_element_type=jnp.float32)
    logits = logits + b_gate_ref[...][None, :]
    iota_e = jax.lax.broadcasted_iota(jnp.int32, (BT, E), 1)
    m1 = jnp.max(logits, axis=1, keepdims=True)
    is1 = (logits == m1)
    idx1 = jnp.min(jnp.where(is1, iota_e, E), axis=1, keepdims=True)
    masked = jnp.where(iota_e == idx1, NEG, logits)
    m2 = jnp.max(masked, axis=1, keepdims=True)
    is2 = (masked == m2)
    idx2 = jnp.min(jnp.where(is2, iota_e, E), axis=1, keepdims=True)
    w1 = 1.0 / (1.0 + jnp.exp(m2 - m1))
    w2 = 1.0 - w1
    eid_ref[...] = jnp.concatenate([idx1[None], idx2[None]], axis=0)
    ws_ref[...] = jnp.concatenate([w1[None], w2[None]], axis=0)


def _tile8(v):
    return jnp.concatenate([v] * 8, axis=1)


def _stage2_kernel(xp_ref, eid_ref, ws_ref, Wih0T_ref, Whh0s_ref, bih0_ref,
                   bhh0_ref, W1s_ref, b1_ref, Wh1s_ref, bh1_ref, Wh2_ref,
                   bh2_ref, out_ref, h0_ref, h1_ref):
    c = pl.program_id(0)
    eidc = eid_ref[...]
    oh_e = (jax.lax.broadcasted_iota(jnp.int32, (R, E), 1)
            == eidc).astype(jnp.float32)
    m0 = (jax.lax.broadcasted_iota(jnp.int32, (R, E * H), 1) // H
          == eidc).astype(jnp.float32)
    m1 = (jax.lax.broadcasted_iota(jnp.int32, (R, E * 2 * H), 1) // (2 * H)
          == eidc).astype(jnp.float32)

    b0i = jnp.dot(oh_e, bih0_ref[...], preferred_element_type=jnp.float32)
    b0h = jnp.dot(oh_e, bhh0_ref[...], preferred_element_type=jnp.float32)
    b1r = jnp.dot(oh_e, b1_ref[...], preferred_element_type=jnp.float32)
    bh1r = jnp.dot(oh_e, bh1_ref[...], preferred_element_type=jnp.float32)
    wh2r = jnp.dot(oh_e, Wh2_ref[...], preferred_element_type=jnp.float32)
    bh2r = jnp.dot(oh_e, bh2_ref[...], preferred_element_type=jnp.float32)

    sel0 = [oh_e[:B, e].reshape(B, 1, 1) for e in range(E)]
    sel1 = [oh_e[B:, e].reshape(B, 1, 1) for e in range(E)]

    Whh0s = Whh0s_ref[...]
    W1s = W1s_ref[...]

    @pl.when(c == 0)
    def _():
        h0_ref[...] = jnp.zeros((R, H), jnp.float32)
        h1_ref[...] = jnp.zeros((R, H), jnp.float32)

    h0 = h0_ref[...]
    h1 = h1_ref[...]

    xf = xp_ref[...].reshape(B * TC, D)
    gs0 = jnp.zeros((B, TC, 3 * H), jnp.float32)
    gs1 = jnp.zeros((B, TC, 3 * H), jnp.float32)
    for e in range(E):
        ge = jnp.dot(xf, Wih0T_ref[e],
                     preferred_element_type=jnp.float32)
        ge = ge.reshape(B, TC, 3 * H)
        gs0 = gs0 + ge * sel0[e]
        gs1 = gs1 + ge * sel1[e]
    gs0 = gs0 + b0i[:B][:, None, :]
    gs1 = gs1 + b0i[B:][:, None, :]

    for i in range(TC):
        gi0 = jnp.concatenate([gs0[:, i, :], gs1[:, i, :]], axis=0)
        gh0 = jnp.dot(_tile8(h0) * m0, Whh0s,
                      preferred_element_type=jnp.float32) + b0h
        cat1 = jnp.concatenate([h0, h1], axis=1)
        g1 = jnp.dot(_tile8(cat1) * m1, W1s,
                     preferred_element_type=jnp.float32) + b1r

        r = jax.nn.sigmoid(gi0[:, :H] + gh0[:, :H])
        z = jax.nn.sigmoid(gi0[:, H:2 * H] + gh0[:, H:2 * H])
        n = jnp.tanh(gi0[:, 2 * H:] + r * gh0[:, 2 * H:])
        h0 = (1.0 - z) * n + z * h0

        r1 = jax.nn.sigmoid(g1[:, :H])
        z1 = jax.nn.sigmoid(g1[:, H:2 * H])
        n1 = jnp.tanh(g1[:, 2 * H:3 * H] + r1 * g1[:, 3 * H:])
        h1n = (1.0 - z1) * n1 + z1 * h1
        if i == 0:
            h1 = h1n * (c > 0).astype(jnp.float32)
        else:
            h1 = h1n

    h0_ref[...] = h0
    h1_ref[...] = h1

    @pl.when(c == NCH - 1)
    def _():
        cat1 = jnp.concatenate([h0, h1], axis=1)
        g1 = jnp.dot(_tile8(cat1) * m1, W1s,
                     preferred_element_type=jnp.float32) + b1r
        r1 = jax.nn.sigmoid(g1[:, :H])
        z1 = jax.nn.sigmoid(g1[:, H:2 * H])
        n1 = jnp.tanh(g1[:, 2 * H:3 * H] + r1 * g1[:, 3 * H:])
        h1f = (1.0 - z1) * n1 + z1 * h1

        zh = jnp.dot(_tile8(h1f) * m0, Wh1s_ref[...],
                     preferred_element_type=jnp.float32) + bh1r
        zh_r = jnp.maximum(zh, 0.0)
        pred = jnp.sum(zh_r * wh2r, axis=1, keepdims=True) + bh2r
        ws = ws_ref[...]
        out_ref[...] = (ws[:B] * pred[:B]) + (ws[B:] * pred[B:])


@jax.jit
def kernel(x, horizon, W_in, b_in, emb, W_gate, b_gate, W_ih0, W_hh0, b_ih0,
           b_hh0, W_ih1, W_hh1, b_ih1, b_hh1, W_h1, b_h1, W_h2, b_h2):
    x = x.astype(jnp.float32)
    horizon = horizon.astype(jnp.int32)

    xp, eid, ws = pl.pallas_call(
        _stage1_kernel,
        grid=(B // BT,),
        in_specs=[
            pl.BlockSpec((BT, L, F), lambda i: (i, 0, 0)),
            pl.BlockSpec((BT,), lambda i: (i,)),
            pl.BlockSpec((D, F), lambda i: (0, 0)),
            pl.BlockSpec((D,), lambda i: (0,)),
            pl.BlockSpec((VOCAB, D), lambda i: (0, 0)),
            pl.BlockSpec((E, D), lambda i: (0, 0)),
            pl.BlockSpec((E,), lambda i: (0,)),
        ],
        out_specs=[
            pl.BlockSpec((BT, L, D), lambda i: (i, 0, 0)),
            pl.BlockSpec((2, BT, 1), lambda i: (0, i, 0)),
            pl.BlockSpec((2, BT, 1), lambda i: (0, i, 0)),
        ],
        out_shape=[
            jax.ShapeDtypeStruct((B, L, D), jnp.float32),
            jax.ShapeDtypeStruct((2, B, 1), jnp.int32),
            jax.ShapeDtypeStruct((2, B, 1), jnp.float32),
        ],
    )(x, horizon, W_in, b_in, emb, W_gate, b_gate)

    Wih0T = W_ih0.transpose(0, 2, 1)
    Whh0s = W_hh0.transpose(0, 2, 1).reshape(E * H, 3 * H)
    Wih1T = W_ih1.transpose(0, 2, 1)
    Whh1T = W_hh1.transpose(0, 2, 1)
    zH = jnp.zeros((E, H, H), jnp.float32)
    top = jnp.concatenate([Wih1T[:, :, :2 * H], Wih1T[:, :, 2 * H:], zH], 2)
    bot = jnp.concatenate([Whh1T[:, :, :2 * H], zH, Whh1T[:, :, 2 * H:]], 2)
    W1s = jnp.concatenate([top, bot], axis=1).reshape(E * 2 * H, 4 * H)
    b1 = jnp.concatenate([b_ih1[:, :2 * H] + b_hh1[:, :2 * H],
                          b_ih1[:, 2 * H:], b_hh1[:, 2 * H:]], axis=1)
    Wh1s = W_h1.transpose(0, 2, 1).reshape(E * H, HEAD)

    out = pl.pallas_call(
        _stage2_kernel,
        grid=(NCH,),
        in_specs=[
            pl.BlockSpec((B, TC, D), lambda c: (0, c, 0)),
            pl.BlockSpec((R, 1), lambda c: (0, 0)),
            pl.BlockSpec((R, 1), lambda c: (0, 0)),
            pl.BlockSpec((E, D, 3 * H), lambda c: (0, 0, 0)),
            pl.BlockSpec((E * H, 3 * H), lambda c: (0, 0)),
            pl.BlockSpec((E, 3 * H), lambda c: (0, 0)),
            pl.BlockSpec((E, 3 * H), lambda c: (0, 0)),
            pl.BlockSpec((E * 2 * H, 4 * H), lambda c: (0, 0)),
            pl.BlockSpec((E, 4 * H), lambda c: (0, 0)),
            pl.BlockSpec((E * H, HEAD), lambda c: (0, 0)),
            pl.BlockSpec((E, HEAD), lambda c: (0, 0)),
            pl.BlockSpec((E, HEAD), lambda c: (0, 0)),
            pl.BlockSpec((E, 1), lambda c: (0, 0)),
        ],
        out_specs=pl.BlockSpec((B, 1), lambda c: (0, 0)),
        out_shape=jax.ShapeDtypeStruct((B, 1), jnp.float32),
        scratch_shapes=[pltpu.VMEM((R, H), jnp.float32),
                        pltpu.VMEM((R, H), jnp.float32)],
    )(xp, eid.reshape(R, 1), ws.reshape(R, 1), Wih0T, Whh0s, b_ih0, b_hh0,
      W1s, b1, Wh1s, b_h1, W_h2.reshape(E, HEAD), b_h2)

    return out[:, 0]
